# baseline (speedup 1.0000x reference)
import functools

import jax
import jax.numpy as jnp
from jax import lax
from jax.experimental import pallas as pl
from jax.experimental.pallas import tpu as pltpu
from jax.experimental.pallas import tpu_sc as plsc

N = 10000
NP = 10240
E_TRAIN = 640000
E_PAD = 655360
E_DEC = 100000
ED_PAD = 102400
HID = 768
NC = 5
CH = 128
NCHUNK = E_PAD // CH
NCHUNK_HALF = NCHUNK // 2
TCH = NCHUNK_HALF // 16
NDCH = ED_PAD // CH
DCH_W = NDCH // 32
NSUB = 16
SLC = NP // NSUB

_mesh = plsc.VectorSubcoreMesh(core_axis_name="c", subcore_axis_name="s")
_sc_params = pltpu.CompilerParams(needs_layout_passes=False,
                                  use_tc_tiling_on_sc=False)


def _fill_const(ref, n16, value):
    @pl.loop(0, n16)
    def _(i):
        ref[pl.ds(i * 16, 16)] = jnp.full((16,), value, jnp.float32)


@functools.partial(
    pl.kernel,
    out_type=jax.ShapeDtypeStruct((2, NP), jnp.float32),
    mesh=_mesh,
    compiler_params=_sc_params,
    scratch_types=[
        pltpu.VMEM((8, CH), jnp.int32),
        pltpu.VMEM((CH,), jnp.float32),
        pltpu.VMEM((SLC,), jnp.float32),
        pltpu.VMEM_SHARED((NP,), jnp.float32),
        pltpu.SemaphoreType.DMA,
    ],
)
def _sc_deg(td2d, deg_out, idx2, ones_v, fill_v, deg_s, sem):
    c = lax.axis_index("c")
    s = lax.axis_index("s")
    base = s * SLC
    start = c * NCHUNK_HALF + s * TCH

    _fill_const(fill_v, SLC // 16, 0.0)
    pltpu.sync_copy(fill_v, deg_s.at[pl.ds(base, SLC)])
    _fill_const(ones_v, CH // 16, 1.0)
    plsc.subcore_barrier()

    @pl.loop(0, TCH // 8)
    def _(b):
        cb = start + b * 8
        pltpu.sync_copy(td2d.at[pl.ds(cb, 8)], idx2)
        descs = [pltpu.async_copy(ones_v, deg_s.at[idx2.at[j]], sem, add=True)
                 for j in range(8)]
        for d in descs:
            d.wait()

    plsc.subcore_barrier()
    pltpu.sync_copy(deg_s.at[pl.ds(base, SLC)], deg_out.at[c, pl.ds(base, SLC)])


@functools.partial(
    pl.kernel,
    out_type=jax.ShapeDtypeStruct((2, NP), jnp.float32),
    mesh=_mesh,
    compiler_params=_sc_params,
    scratch_types=[
        pltpu.VMEM((8, CH), jnp.int32),
        pltpu.VMEM((8, CH), jnp.int32),
        pltpu.VMEM((8, CH), jnp.float32),
        pltpu.VMEM((SLC,), jnp.float32),
        pltpu.VMEM((NP,), jnp.float32),
        pltpu.VMEM_SHARED((NP,), jnp.float32),
        pltpu.SemaphoreType.DMA,
    ],
)
def _sc_g1(ts2d, td2d, u_hbm, g1_out, idxa2, idxb2, valb, fill_v, tab, g1_s,
           sem):
    c = lax.axis_index("c")
    s = lax.axis_index("s")
    base = s * SLC
    start = c * NCHUNK_HALF + s * TCH

    _fill_const(fill_v, SLC // 16, 0.0)
    pltpu.sync_copy(fill_v, g1_s.at[pl.ds(base, SLC)])
    pltpu.sync_copy(u_hbm, tab)
    plsc.subcore_barrier()

    @pl.loop(0, TCH // 8)
    def _(b):
        cb = start + b * 8
        pltpu.sync_copy(ts2d.at[pl.ds(cb, 8)], idxa2)
        pltpu.sync_copy(td2d.at[pl.ds(cb, 8)], idxb2)

        @pl.loop(0, 8)
        def _(r):
            for k in range(CH // 16):
                sl = pl.ds(k * 16, 16)
                valb[r, sl] = plsc.load_gather(tab, [idxa2[r, sl]])

        descs = [pltpu.async_copy(valb.at[j], g1_s.at[idxb2.at[j]], sem,
                                  add=True)
                 for j in range(8)]
        for d in descs:
            d.wait()

    plsc.subcore_barrier()
    pltpu.sync_copy(g1_s.at[pl.ds(base, SLC)], g1_out.at[c, pl.ds(base, SLC)])


@functools.partial(
    pl.kernel,
    out_type=jax.ShapeDtypeStruct((2, NSUB, 64, NP), jnp.float32),
    mesh=_mesh,
    compiler_params=_sc_params,
    scratch_types=[
        pltpu.VMEM((TCH, CH), jnp.int32),
        pltpu.VMEM((TCH, CH), jnp.int32),
        pltpu.VMEM((NP,), jnp.float32),
        pltpu.VMEM((NP,), jnp.float32),
        pltpu.VMEM((NP,), jnp.float32),
        pltpu.VMEM((NP,), jnp.float32),
        pltpu.VMEM((NP,), jnp.float32),
        pltpu.VMEM((NP,), jnp.float32),
        pltpu.VMEM((NP,), jnp.float32),
        pltpu.VMEM((NP,), jnp.float32),
        pltpu.SemaphoreType.DMA,
        pltpu.SemaphoreType.DMA,
        pltpu.SemaphoreType.DMA,
        pltpu.SemaphoreType.DMA,
    ],
)
def _sc_stage2(ts2d, td2d, u2T, out, its_r, itd_r,
               tab0A, tab1A, tab0B, tab1B, acc0A, acc1A, acc0B, acc1B,
               tsA, tsB, osA, osB):
    c = lax.axis_index("c")
    s = lax.axis_index("s")
    start = c * NCHUNK_HALF + s * TCH

    pltpu.sync_copy(ts2d.at[pl.ds(start, TCH)], its_r)
    pltpu.sync_copy(td2d.at[pl.ds(start, TCH)], itd_r)

    pltpu.async_copy(u2T.at[0], tab0A, tsA)
    pltpu.async_copy(u2T.at[1], tab1A, tsA)

    def do_pair(tab0, tab1, acc0, acc1):
        _fill_const(acc0, NP // 16, 0.0)
        _fill_const(acc1, NP // 16, 0.0)

        @pl.loop(0, TCH)
        def _(ch):
            for j in range(8):
                sl = pl.ds(j * 16, 16)
                its = its_r[ch, sl]
                itd = itd_r[ch, sl]
                v0 = plsc.load_gather(tab0, [its])
                plsc.addupdate_scatter(acc0, [itd], v0)
                v1 = plsc.load_gather(tab1, [its])
                plsc.addupdate_scatter(acc1, [itd], v1)

    @pl.loop(0, 16)
    def _(k):
        colA = 4 * k
        colB = 4 * k + 2

        pltpu.make_async_copy(u2T.at[colA], tab0A, tsA).wait()
        pltpu.make_async_copy(u2T.at[colA + 1], tab1A, tsA).wait()
        pltpu.async_copy(u2T.at[colB], tab0B, tsB)
        pltpu.async_copy(u2T.at[colB + 1], tab1B, tsB)

        @pl.when(k > 0)
        def _():
            pltpu.make_async_copy(acc0A, out.at[c, s, colA - 4], osA).wait()
            pltpu.make_async_copy(acc1A, out.at[c, s, colA - 3], osA).wait()

        do_pair(tab0A, tab1A, acc0A, acc1A)
        pltpu.async_copy(acc0A, out.at[c, s, colA], osA)
        pltpu.async_copy(acc1A, out.at[c, s, colA + 1], osA)

        pltpu.make_async_copy(u2T.at[colB], tab0B, tsB).wait()
        pltpu.make_async_copy(u2T.at[colB + 1], tab1B, tsB).wait()

        @pl.when(k < 15)
        def _():
            pltpu.async_copy(u2T.at[colB + 2], tab0A, tsA)
            pltpu.async_copy(u2T.at[colB + 3], tab1A, tsA)

        @pl.when(k > 0)
        def _():
            pltpu.make_async_copy(acc0B, out.at[c, s, colB - 4], osB).wait()
            pltpu.make_async_copy(acc1B, out.at[c, s, colB - 3], osB).wait()

        do_pair(tab0B, tab1B, acc0B, acc1B)
        pltpu.async_copy(acc0B, out.at[c, s, colB], osB)
        pltpu.async_copy(acc1B, out.at[c, s, colB + 1], osB)

    pltpu.make_async_copy(acc0A, out.at[c, s, 60], osA).wait()
    pltpu.make_async_copy(acc1A, out.at[c, s, 61], osA).wait()
    pltpu.make_async_copy(acc0B, out.at[c, s, 62], osB).wait()
    pltpu.make_async_copy(acc1B, out.at[c, s, 63], osB).wait()


@functools.partial(
    pl.kernel,
    out_type=jax.ShapeDtypeStruct((ED_PAD, 64), jnp.float32),
    mesh=_mesh,
    compiler_params=_sc_params,
    scratch_types=[
        pltpu.VMEM((DCH_W, CH), jnp.int32),
        pltpu.VMEM((DCH_W, CH), jnp.int32),
        pltpu.VMEM((5, CH, 64), jnp.float32),
        pltpu.VMEM((5, CH, 64), jnp.float32),
        pltpu.SemaphoreType.DMA,
        pltpu.SemaphoreType.DMA,
        pltpu.SemaphoreType.DMA,
    ],
)
def _sc_stage3(e0_2d, e1_2d, z2_hbm, nr_out, idxa, idxb, rows0, rows1,
               g0sem, g1sem, stsem):
    c = lax.axis_index("c")
    s = lax.axis_index("s")
    wid = s * 2 + c
    start = wid * DCH_W
    nb = DCH_W // 5

    pltpu.sync_copy(e0_2d.at[pl.ds(start, DCH_W)], idxa)
    pltpu.sync_copy(e1_2d.at[pl.ds(start, DCH_W)], idxb)

    def fire_gathers(batch):
        for j in range(5):
            pltpu.async_copy(z2_hbm.at[idxa.at[batch * 5 + j]],
                             rows0.at[j], g0sem)
            pltpu.async_copy(z2_hbm.at[idxb.at[batch * 5 + j]],
                             rows1.at[j], g1sem)

    def wait_gathers():
        for j in range(5):
            pltpu.make_async_copy(z2_hbm.at[idxa.at[j]], rows0.at[j],
                                  g0sem).wait()
            pltpu.make_async_copy(z2_hbm.at[idxb.at[j]], rows1.at[j],
                                  g1sem).wait()

    fire_gathers(0)

    @pl.loop(0, nb)
    def _(b):
        wait_gathers()

        @pl.loop(0, CH)
        def _(r):
            for j in range(5):
                for k in range(4):
                    sl = pl.ds(k * 16, 16)
                    rows0[j, r, sl] = rows0[j, r, sl] * rows1[j, r, sl]

        for j in range(5):
            pltpu.async_copy(
                rows0.at[j],
                nr_out.at[pl.ds((start + b * 5 + j) * CH, CH)], stsem)

        for j in range(5):
            pltpu.make_async_copy(rows0.at[j], nr_out.at[pl.ds(0, CH)],
                                  stsem).wait()

        @pl.when(b < nb - 1)
        def _():
            fire_gathers(b + 1)


def _tc_prep_body(dega_ref, degb_ref, x_ref, dinv_ref, u_ref):
    deg = dega_ref[...] + degb_ref[...] + 1.0
    dinv = lax.rsqrt(jnp.maximum(deg, 1e-12))
    dinv_ref[...] = dinv
    u_ref[...] = x_ref[...] * dinv


def _tc_mid_body(dinv_ref, u_ref, g1a_ref, g1b_ref, W1_ref, b1_ref, W2_ref,
                 u2_ref, u2T_ref):
    dinv = dinv_ref[...]
    u = u_ref[...]
    s1 = dinv * (g1a_ref[...] + g1b_ref[...] + u)
    z1 = jnp.maximum(s1 * W1_ref[...] + b1_ref[...], 0.0)
    h2 = jnp.dot(z1, W2_ref[...], preferred_element_type=jnp.float32)
    u2 = h2 * dinv
    u2_ref[...] = u2
    u2T_ref[...] = u2.T


def _tc_z2_body(dinv_ref, p_ref, u2_ref, b2_ref, z2_ref):
    dinv = dinv_ref[...]
    g2 = jnp.sum(p_ref[...], axis=(0, 1))
    agg = dinv * (g2.T + u2_ref[...])
    z2_ref[...] = jnp.maximum(agg + b2_ref[...], 0.0)


def _tc_dec_body(nr_ref, ea_ref, L1n_ref, L1a_ref, L1b_ref, L2w_ref, L2b_ref,
                 out_ref):
    a = jnp.dot(nr_ref[...], L1n_ref[...], preferred_element_type=jnp.float32)
    a += jnp.dot(ea_ref[...], L1a_ref[...], preferred_element_type=jnp.float32)
    h = jnp.maximum(a + L1b_ref[...], 0.0)
    logits = jnp.dot(h, L2w_ref[...], preferred_element_type=jnp.float32)
    logits += L2b_ref[...]
    m = jnp.max(logits, axis=-1, keepdims=True)
    e = jnp.exp(logits - m)
    out_ref[...] = e / jnp.sum(e, axis=-1, keepdims=True)


_EB = 2048


def kernel(x, train_edge_index, edge_index, edge_attr,
           W1, b1, W2, b2, L1w, L1b, L2w, L2b):
    f32 = jnp.float32
    npad = E_PAD - E_TRAIN
    ts2d = jnp.concatenate(
        [train_edge_index[0],
         jnp.zeros((npad,), jnp.int32)]).reshape(NCHUNK, CH)
    td2d = jnp.concatenate(
        [train_edge_index[1],
         jnp.full((npad,), NP - 1, jnp.int32)]).reshape(NCHUNK, CH)
    e0_2d = jnp.pad(edge_index[0], (0, ED_PAD - E_DEC)).reshape(NDCH, CH)
    e1_2d = jnp.pad(edge_index[1], (0, ED_PAD - E_DEC)).reshape(NDCH, CH)
    xp = jnp.pad(x[:, 0], (0, NP - N))

    deg = _sc_deg(td2d)

    dinv2, u = pl.pallas_call(
        _tc_prep_body,
        out_shape=(jax.ShapeDtypeStruct((NP, 1), f32),
                   jax.ShapeDtypeStruct((NP, 1), f32)),
    )(deg[0].reshape(NP, 1), deg[1].reshape(NP, 1), xp.reshape(NP, 1))

    g1 = _sc_g1(ts2d, td2d, u.reshape(NP))

    u2, u2T = pl.pallas_call(
        _tc_mid_body,
        out_shape=(jax.ShapeDtypeStruct((NP, 64), f32),
                   jax.ShapeDtypeStruct((64, NP), f32)),
    )(dinv2, u, g1[0].reshape(NP, 1), g1[1].reshape(NP, 1),
      W1, b1.reshape(1, 128), W2)

    g2p = _sc_stage2(ts2d, td2d, u2T)

    _NB = 2048
    z2 = pl.pallas_call(
        _tc_z2_body,
        grid=(NP // _NB,),
        in_specs=[
            pl.BlockSpec((_NB, 1), lambda i: (i, 0)),
            pl.BlockSpec((2, NSUB, 64, _NB), lambda i: (0, 0, 0, i)),
            pl.BlockSpec((_NB, 64), lambda i: (i, 0)),
            pl.BlockSpec((1, 64), lambda i: (0, 0)),
        ],
        out_specs=pl.BlockSpec((_NB, 64), lambda i: (i, 0)),
        out_shape=jax.ShapeDtypeStruct((NP, 64), f32),
    )(dinv2, g2p, u2, b2.reshape(1, 64))

    nr = _sc_stage3(e0_2d, e1_2d, z2)

    grid = (E_DEC + _EB - 1) // _EB
    out = pl.pallas_call(
        _tc_dec_body,
        grid=(grid,),
        in_specs=[
            pl.BlockSpec((_EB, 64), lambda i: (i, 0)),
            pl.BlockSpec((_EB, HID), lambda i: (i, 0)),
            pl.BlockSpec((64, 128), lambda i: (0, 0)),
            pl.BlockSpec((HID, 128), lambda i: (0, 0)),
            pl.BlockSpec((1, 128), lambda i: (0, 0)),
            pl.BlockSpec((128, NC), lambda i: (0, 0)),
            pl.BlockSpec((1, NC), lambda i: (0, 0)),
        ],
        out_specs=pl.BlockSpec((_EB, NC), lambda i: (i, 0)),
        out_shape=jax.ShapeDtypeStruct((E_DEC, NC), f32),
    )(nr, edge_attr, L1w[:64], L1w[64:], L1b.reshape(1, 128),
      L2w, L2b.reshape(1, NC))

    return out


# speedup vs baseline: 1.0884x; 1.0884x over previous
import functools

import jax
import jax.numpy as jnp
from jax import lax
from jax.experimental import pallas as pl
from jax.experimental.pallas import tpu as pltpu
from jax.experimental.pallas import tpu_sc as plsc

N = 10000
NP = 10240
E_TRAIN = 640000
E_PAD = 655360
E_DEC = 100000
ED_PAD = 102400
HID = 768
NC = 5
CH = 128
NCHUNK = E_PAD // CH
NCHUNK_HALF = NCHUNK // 2
TCH = NCHUNK_HALF // 16
NDCH = ED_PAD // CH
DCH_W = NDCH // 32
NSUB = 16
SLC = NP // NSUB

_mesh = plsc.VectorSubcoreMesh(core_axis_name="c", subcore_axis_name="s")
_sc_params = pltpu.CompilerParams(needs_layout_passes=False,
                                  use_tc_tiling_on_sc=False)


def _fill_const(ref, n16, value):
    @pl.loop(0, n16)
    def _(i):
        ref[pl.ds(i * 16, 16)] = jnp.full((16,), value, jnp.float32)


@functools.partial(
    pl.kernel,
    out_type=jax.ShapeDtypeStruct((2, NP), jnp.float32),
    mesh=_mesh,
    compiler_params=_sc_params,
    scratch_types=[
        pltpu.VMEM((8, CH), jnp.int32),
        pltpu.VMEM((CH,), jnp.float32),
        pltpu.VMEM((SLC,), jnp.float32),
        pltpu.VMEM_SHARED((NP,), jnp.float32),
        pltpu.SemaphoreType.DMA,
    ],
)
def _sc_deg(td2d, deg_out, idx2, ones_v, fill_v, deg_s, sem):
    c = lax.axis_index("c")
    s = lax.axis_index("s")
    base = s * SLC
    start = c * NCHUNK_HALF + s * TCH

    _fill_const(fill_v, SLC // 16, 0.0)
    pltpu.sync_copy(fill_v, deg_s.at[pl.ds(base, SLC)])
    _fill_const(ones_v, CH // 16, 1.0)
    plsc.subcore_barrier()

    @pl.loop(0, TCH // 8)
    def _(b):
        cb = start + b * 8
        pltpu.sync_copy(td2d.at[pl.ds(cb, 8)], idx2)
        descs = [pltpu.async_copy(ones_v, deg_s.at[idx2.at[j]], sem, add=True)
                 for j in range(8)]
        for d in descs:
            d.wait()

    plsc.subcore_barrier()
    pltpu.sync_copy(deg_s.at[pl.ds(base, SLC)], deg_out.at[c, pl.ds(base, SLC)])


@functools.partial(
    pl.kernel,
    out_type=jax.ShapeDtypeStruct((2, NP), jnp.float32),
    mesh=_mesh,
    compiler_params=_sc_params,
    scratch_types=[
        pltpu.VMEM((8, CH), jnp.int32),
        pltpu.VMEM((8, CH), jnp.int32),
        pltpu.VMEM((8, CH), jnp.float32),
        pltpu.VMEM((SLC,), jnp.float32),
        pltpu.VMEM((NP,), jnp.float32),
        pltpu.VMEM_SHARED((NP,), jnp.float32),
        pltpu.SemaphoreType.DMA,
    ],
)
def _sc_g1(ts2d, td2d, u_hbm, g1_out, idxa2, idxb2, valb, fill_v, tab, g1_s,
           sem):
    c = lax.axis_index("c")
    s = lax.axis_index("s")
    base = s * SLC
    start = c * NCHUNK_HALF + s * TCH

    _fill_const(fill_v, SLC // 16, 0.0)
    pltpu.sync_copy(fill_v, g1_s.at[pl.ds(base, SLC)])
    pltpu.sync_copy(u_hbm, tab)
    plsc.subcore_barrier()

    @pl.loop(0, TCH // 8)
    def _(b):
        cb = start + b * 8
        pltpu.sync_copy(ts2d.at[pl.ds(cb, 8)], idxa2)
        pltpu.sync_copy(td2d.at[pl.ds(cb, 8)], idxb2)

        @pl.loop(0, 8)
        def _(r):
            for k in range(CH // 16):
                sl = pl.ds(k * 16, 16)
                valb[r, sl] = plsc.load_gather(tab, [idxa2[r, sl]])

        descs = [pltpu.async_copy(valb.at[j], g1_s.at[idxb2.at[j]], sem,
                                  add=True)
                 for j in range(8)]
        for d in descs:
            d.wait()

    plsc.subcore_barrier()
    pltpu.sync_copy(g1_s.at[pl.ds(base, SLC)], g1_out.at[c, pl.ds(base, SLC)])


@functools.partial(
    pl.kernel,
    out_type=jax.ShapeDtypeStruct((2, NSUB, 64, NP), jnp.float32),
    mesh=_mesh,
    compiler_params=_sc_params,
    scratch_types=[
        pltpu.VMEM((TCH, CH), jnp.int32),
        pltpu.VMEM((TCH, CH), jnp.int32),
        pltpu.VMEM((NP,), jnp.float32),
        pltpu.VMEM((NP,), jnp.float32),
        pltpu.VMEM((NP,), jnp.float32),
        pltpu.VMEM((NP,), jnp.float32),
        pltpu.VMEM((NP,), jnp.float32),
        pltpu.VMEM((NP,), jnp.float32),
        pltpu.VMEM((NP,), jnp.float32),
        pltpu.VMEM((NP,), jnp.float32),
        pltpu.SemaphoreType.DMA,
        pltpu.SemaphoreType.DMA,
        pltpu.SemaphoreType.DMA,
        pltpu.SemaphoreType.DMA,
    ],
)
def _sc_stage2(ts2d, td2d, u2T, out, its_r, itd_r,
               tab0A, tab1A, tab0B, tab1B, acc0A, acc1A, acc0B, acc1B,
               tsA, tsB, osA, osB):
    c = lax.axis_index("c")
    s = lax.axis_index("s")
    start = c * NCHUNK_HALF + s * TCH

    pltpu.sync_copy(ts2d.at[pl.ds(start, TCH)], its_r)
    pltpu.sync_copy(td2d.at[pl.ds(start, TCH)], itd_r)

    pltpu.async_copy(u2T.at[0], tab0A, tsA)
    pltpu.async_copy(u2T.at[1], tab1A, tsA)

    def do_pair(tab0, tab1, acc0, acc1):
        _fill_const(acc0, NP // 16, 0.0)
        _fill_const(acc1, NP // 16, 0.0)

        @pl.loop(0, TCH)
        def _(ch):
            its = [its_r[ch, pl.ds(j * 16, 16)] for j in range(8)]
            itd = [itd_r[ch, pl.ds(j * 16, 16)] for j in range(8)]
            v0 = [plsc.load_gather(tab0, [its[j]]) for j in range(8)]
            v1 = [plsc.load_gather(tab1, [its[j]]) for j in range(8)]
            for j in range(8):
                plsc.addupdate_scatter(acc0, [itd[j]], v0[j])
            for j in range(8):
                plsc.addupdate_scatter(acc1, [itd[j]], v1[j])

    @pl.loop(0, 16)
    def _(k):
        colA = 4 * k
        colB = 4 * k + 2

        pltpu.make_async_copy(u2T.at[colA], tab0A, tsA).wait()
        pltpu.make_async_copy(u2T.at[colA + 1], tab1A, tsA).wait()
        pltpu.async_copy(u2T.at[colB], tab0B, tsB)
        pltpu.async_copy(u2T.at[colB + 1], tab1B, tsB)

        @pl.when(k > 0)
        def _():
            pltpu.make_async_copy(acc0A, out.at[c, s, colA - 4], osA).wait()
            pltpu.make_async_copy(acc1A, out.at[c, s, colA - 3], osA).wait()

        do_pair(tab0A, tab1A, acc0A, acc1A)
        pltpu.async_copy(acc0A, out.at[c, s, colA], osA)
        pltpu.async_copy(acc1A, out.at[c, s, colA + 1], osA)

        pltpu.make_async_copy(u2T.at[colB], tab0B, tsB).wait()
        pltpu.make_async_copy(u2T.at[colB + 1], tab1B, tsB).wait()

        @pl.when(k < 15)
        def _():
            pltpu.async_copy(u2T.at[colB + 2], tab0A, tsA)
            pltpu.async_copy(u2T.at[colB + 3], tab1A, tsA)

        @pl.when(k > 0)
        def _():
            pltpu.make_async_copy(acc0B, out.at[c, s, colB - 4], osB).wait()
            pltpu.make_async_copy(acc1B, out.at[c, s, colB - 3], osB).wait()

        do_pair(tab0B, tab1B, acc0B, acc1B)
        pltpu.async_copy(acc0B, out.at[c, s, colB], osB)
        pltpu.async_copy(acc1B, out.at[c, s, colB + 1], osB)

    pltpu.make_async_copy(acc0A, out.at[c, s, 60], osA).wait()
    pltpu.make_async_copy(acc1A, out.at[c, s, 61], osA).wait()
    pltpu.make_async_copy(acc0B, out.at[c, s, 62], osB).wait()
    pltpu.make_async_copy(acc1B, out.at[c, s, 63], osB).wait()


@functools.partial(
    pl.kernel,
    out_type=jax.ShapeDtypeStruct((ED_PAD, 64), jnp.float32),
    mesh=_mesh,
    compiler_params=_sc_params,
    scratch_types=[
        pltpu.VMEM((DCH_W, CH), jnp.int32),
        pltpu.VMEM((DCH_W, CH), jnp.int32),
        pltpu.VMEM((5, CH, 64), jnp.float32),
        pltpu.VMEM((5, CH, 64), jnp.float32),
        pltpu.SemaphoreType.DMA,
        pltpu.SemaphoreType.DMA,
        pltpu.SemaphoreType.DMA,
    ],
)
def _sc_stage3(e0_2d, e1_2d, z2_hbm, nr_out, idxa, idxb, rows0, rows1,
               g0sem, g1sem, stsem):
    c = lax.axis_index("c")
    s = lax.axis_index("s")
    wid = s * 2 + c
    start = wid * DCH_W
    nb = DCH_W // 5

    pltpu.sync_copy(e0_2d.at[pl.ds(start, DCH_W)], idxa)
    pltpu.sync_copy(e1_2d.at[pl.ds(start, DCH_W)], idxb)

    def fire_gathers(batch):
        for j in range(5):
            pltpu.async_copy(z2_hbm.at[idxa.at[batch * 5 + j]],
                             rows0.at[j], g0sem)
            pltpu.async_copy(z2_hbm.at[idxb.at[batch * 5 + j]],
                             rows1.at[j], g1sem)

    def wait_gathers():
        for j in range(5):
            pltpu.make_async_copy(z2_hbm.at[idxa.at[j]], rows0.at[j],
                                  g0sem).wait()
            pltpu.make_async_copy(z2_hbm.at[idxb.at[j]], rows1.at[j],
                                  g1sem).wait()

    fire_gathers(0)

    @pl.loop(0, nb)
    def _(b):
        wait_gathers()

        @pl.loop(0, CH)
        def _(r):
            for j in range(5):
                for k in range(4):
                    sl = pl.ds(k * 16, 16)
                    rows0[j, r, sl] = rows0[j, r, sl] * rows1[j, r, sl]

        for j in range(5):
            pltpu.async_copy(
                rows0.at[j],
                nr_out.at[pl.ds((start + b * 5 + j) * CH, CH)], stsem)

        for j in range(5):
            pltpu.make_async_copy(rows0.at[j], nr_out.at[pl.ds(0, CH)],
                                  stsem).wait()

        @pl.when(b < nb - 1)
        def _():
            fire_gathers(b + 1)


def _tc_prep_body(dega_ref, degb_ref, x_ref, dinv_ref, u_ref):
    deg = dega_ref[...] + degb_ref[...] + 1.0
    dinv = lax.rsqrt(jnp.maximum(deg, 1e-12))
    dinv_ref[...] = dinv
    u_ref[...] = x_ref[...] * dinv


def _tc_mid_body(dinv_ref, u_ref, g1a_ref, g1b_ref, W1_ref, b1_ref, W2_ref,
                 u2_ref, u2T_ref):
    dinv = dinv_ref[...]
    u = u_ref[...]
    s1 = dinv * (g1a_ref[...] + g1b_ref[...] + u)
    z1 = jnp.maximum(s1 * W1_ref[...] + b1_ref[...], 0.0)
    h2 = jnp.dot(z1, W2_ref[...], preferred_element_type=jnp.float32)
    u2 = h2 * dinv
    u2_ref[...] = u2
    u2T_ref[...] = u2.T


def _tc_z2_body(dinv_ref, p_ref, u2_ref, b2_ref, z2_ref):
    dinv = dinv_ref[...]
    g2 = jnp.sum(p_ref[...], axis=(0, 1))
    agg = dinv * (g2.T + u2_ref[...])
    z2_ref[...] = jnp.maximum(agg + b2_ref[...], 0.0)


def _tc_dec_body(nr_ref, ea_ref, L1n_ref, L1a_ref, L1b_ref, L2w_ref, L2b_ref,
                 out_ref):
    a = jnp.dot(nr_ref[...], L1n_ref[...], preferred_element_type=jnp.float32)
    a += jnp.dot(ea_ref[...], L1a_ref[...], preferred_element_type=jnp.float32)
    h = jnp.maximum(a + L1b_ref[...], 0.0)
    logits = jnp.dot(h, L2w_ref[...], preferred_element_type=jnp.float32)
    logits += L2b_ref[...]
    m = jnp.max(logits, axis=-1, keepdims=True)
    e = jnp.exp(logits - m)
    out_ref[...] = e / jnp.sum(e, axis=-1, keepdims=True)


_EB = 2048


def kernel(x, train_edge_index, edge_index, edge_attr,
           W1, b1, W2, b2, L1w, L1b, L2w, L2b):
    f32 = jnp.float32
    npad = E_PAD - E_TRAIN
    ts2d = jnp.concatenate(
        [train_edge_index[0],
         jnp.zeros((npad,), jnp.int32)]).reshape(NCHUNK, CH)
    td2d = jnp.concatenate(
        [train_edge_index[1],
         jnp.full((npad,), NP - 1, jnp.int32)]).reshape(NCHUNK, CH)
    e0_2d = jnp.pad(edge_index[0], (0, ED_PAD - E_DEC)).reshape(NDCH, CH)
    e1_2d = jnp.pad(edge_index[1], (0, ED_PAD - E_DEC)).reshape(NDCH, CH)
    xp = jnp.pad(x[:, 0], (0, NP - N))

    deg = _sc_deg(td2d)

    dinv2, u = pl.pallas_call(
        _tc_prep_body,
        out_shape=(jax.ShapeDtypeStruct((NP, 1), f32),
                   jax.ShapeDtypeStruct((NP, 1), f32)),
    )(deg[0].reshape(NP, 1), deg[1].reshape(NP, 1), xp.reshape(NP, 1))

    g1 = _sc_g1(ts2d, td2d, u.reshape(NP))

    u2, u2T = pl.pallas_call(
        _tc_mid_body,
        out_shape=(jax.ShapeDtypeStruct((NP, 64), f32),
                   jax.ShapeDtypeStruct((64, NP), f32)),
    )(dinv2, u, g1[0].reshape(NP, 1), g1[1].reshape(NP, 1),
      W1, b1.reshape(1, 128), W2)

    g2p = _sc_stage2(ts2d, td2d, u2T)

    _NB = 2048
    z2 = pl.pallas_call(
        _tc_z2_body,
        grid=(NP // _NB,),
        in_specs=[
            pl.BlockSpec((_NB, 1), lambda i: (i, 0)),
            pl.BlockSpec((2, NSUB, 64, _NB), lambda i: (0, 0, 0, i)),
            pl.BlockSpec((_NB, 64), lambda i: (i, 0)),
            pl.BlockSpec((1, 64), lambda i: (0, 0)),
        ],
        out_specs=pl.BlockSpec((_NB, 64), lambda i: (i, 0)),
        out_shape=jax.ShapeDtypeStruct((NP, 64), f32),
    )(dinv2, g2p, u2, b2.reshape(1, 64))

    nr = _sc_stage3(e0_2d, e1_2d, z2)

    grid = (E_DEC + _EB - 1) // _EB
    out = pl.pallas_call(
        _tc_dec_body,
        grid=(grid,),
        in_specs=[
            pl.BlockSpec((_EB, 64), lambda i: (i, 0)),
            pl.BlockSpec((_EB, HID), lambda i: (i, 0)),
            pl.BlockSpec((64, 128), lambda i: (0, 0)),
            pl.BlockSpec((HID, 128), lambda i: (0, 0)),
            pl.BlockSpec((1, 128), lambda i: (0, 0)),
            pl.BlockSpec((128, NC), lambda i: (0, 0)),
            pl.BlockSpec((1, NC), lambda i: (0, 0)),
        ],
        out_specs=pl.BlockSpec((_EB, NC), lambda i: (i, 0)),
        out_shape=jax.ShapeDtypeStruct((E_DEC, NC), f32),
    )(nr, edge_attr, L1w[:64], L1w[64:], L1b.reshape(1, 128),
      L2w, L2b.reshape(1, NC))

    return out


# speedup vs baseline: 1.9889x; 1.8273x over previous
import functools

import jax
import jax.numpy as jnp
from jax import lax
from jax.experimental import pallas as pl
from jax.experimental.pallas import tpu as pltpu
from jax.experimental.pallas import tpu_sc as plsc

N = 10000
NP = 10240
E_TRAIN = 640000
E_PAD = 655360
E_DEC = 100000
ED_PAD = 102400
HID = 768
NC = 5
CH = 128
NCHUNK = E_PAD // CH
NCHUNK_HALF = NCHUNK // 2
TCH = NCHUNK_HALF // 16
NDCH = ED_PAD // CH
DCH_W = NDCH // 32
NSUB = 16
SLC = NP // NSUB

_mesh = plsc.VectorSubcoreMesh(core_axis_name="c", subcore_axis_name="s")
_sc_params = pltpu.CompilerParams(needs_layout_passes=False,
                                  use_tc_tiling_on_sc=False)


def _fill_const(ref, n16, value):
    @pl.loop(0, n16)
    def _(i):
        ref[pl.ds(i * 16, 16)] = jnp.full((16,), value, jnp.float32)


@functools.partial(
    pl.kernel,
    out_type=jax.ShapeDtypeStruct((2, NP), jnp.float32),
    mesh=_mesh,
    compiler_params=_sc_params,
    scratch_types=[
        pltpu.VMEM((8, CH), jnp.int32),
        pltpu.VMEM((CH,), jnp.float32),
        pltpu.VMEM((SLC,), jnp.float32),
        pltpu.VMEM_SHARED((NP,), jnp.float32),
        pltpu.SemaphoreType.DMA,
    ],
)
def _sc_deg(td2d, deg_out, idx2, ones_v, fill_v, deg_s, sem):
    c = lax.axis_index("c")
    s = lax.axis_index("s")
    base = s * SLC
    start = c * NCHUNK_HALF + s * TCH

    _fill_const(fill_v, SLC // 16, 0.0)
    pltpu.sync_copy(fill_v, deg_s.at[pl.ds(base, SLC)])
    _fill_const(ones_v, CH // 16, 1.0)
    plsc.subcore_barrier()

    @pl.loop(0, TCH // 8)
    def _(b):
        cb = start + b * 8
        pltpu.sync_copy(td2d.at[pl.ds(cb, 8)], idx2)
        descs = [pltpu.async_copy(ones_v, deg_s.at[idx2.at[j]], sem, add=True)
                 for j in range(8)]
        for d in descs:
            d.wait()

    plsc.subcore_barrier()
    pltpu.sync_copy(deg_s.at[pl.ds(base, SLC)], deg_out.at[c, pl.ds(base, SLC)])


@functools.partial(
    pl.kernel,
    out_type=jax.ShapeDtypeStruct((2, NP), jnp.float32),
    mesh=_mesh,
    compiler_params=_sc_params,
    scratch_types=[
        pltpu.VMEM((8, CH), jnp.int32),
        pltpu.VMEM((8, CH), jnp.int32),
        pltpu.VMEM((8, CH), jnp.float32),
        pltpu.VMEM((SLC,), jnp.float32),
        pltpu.VMEM((NP,), jnp.float32),
        pltpu.VMEM_SHARED((NP,), jnp.float32),
        pltpu.SemaphoreType.DMA,
    ],
)
def _sc_g1(ts2d, td2d, u_hbm, g1_out, idxa2, idxb2, valb, fill_v, tab, g1_s,
           sem):
    c = lax.axis_index("c")
    s = lax.axis_index("s")
    base = s * SLC
    start = c * NCHUNK_HALF + s * TCH

    _fill_const(fill_v, SLC // 16, 0.0)
    pltpu.sync_copy(fill_v, g1_s.at[pl.ds(base, SLC)])
    pltpu.sync_copy(u_hbm, tab)
    plsc.subcore_barrier()

    @pl.loop(0, TCH // 8)
    def _(b):
        cb = start + b * 8
        pltpu.sync_copy(ts2d.at[pl.ds(cb, 8)], idxa2)
        pltpu.sync_copy(td2d.at[pl.ds(cb, 8)], idxb2)

        @pl.loop(0, 8)
        def _(r):
            for k in range(CH // 16):
                sl = pl.ds(k * 16, 16)
                valb[r, sl] = plsc.load_gather(tab, [idxa2[r, sl]])

        descs = [pltpu.async_copy(valb.at[j], g1_s.at[idxb2.at[j]], sem,
                                  add=True)
                 for j in range(8)]
        for d in descs:
            d.wait()

    plsc.subcore_barrier()
    pltpu.sync_copy(g1_s.at[pl.ds(base, SLC)], g1_out.at[c, pl.ds(base, SLC)])


@functools.partial(
    pl.kernel,
    out_type=jax.ShapeDtypeStruct((2, NP, 64), jnp.float32),
    mesh=_mesh,
    compiler_params=_sc_params,
    scratch_types=[
        pltpu.VMEM((8, CH), jnp.int32),
        pltpu.VMEM((8, CH), jnp.int32),
        pltpu.VMEM((8, CH, 64), jnp.float32),
        pltpu.VMEM_SHARED((NP, 64), jnp.float32),
        pltpu.SemaphoreType.DMA,
        pltpu.SemaphoreType.DMA,
        pltpu.SemaphoreType.DMA,
        pltpu.SemaphoreType.DMA,
    ],
)
def _sc_stage2(ts2d, td2d, u2_hbm, g2_out, idxa, idxb, rows, acc_s,
               gsemA, gsemB, ssemA, ssemB):
    c = lax.axis_index("c")
    s = lax.axis_index("s")
    start = c * NCHUNK_HALF + s * TCH

    @pl.loop(0, CH)
    def _(r):
        for j in range(4):
            rows[0, r, pl.ds(j * 16, 16)] = jnp.zeros((16,), jnp.float32)

    for k in range(SLC // CH):
        pltpu.sync_copy(rows.at[0], acc_s.at[pl.ds(s * SLC + k * CH, CH)])
    plsc.subcore_barrier()

    gsems = (gsemA, gsemB)
    ssems = (ssemA, ssemB)

    def fire_gathers(st):
        for j in range(4):
            pltpu.async_copy(u2_hbm.at[idxa.at[st * 4 + j]],
                             rows.at[st * 4 + j], gsems[st])

    def fire_scatters(st):
        for j in range(4):
            pltpu.async_copy(rows.at[st * 4 + j],
                             acc_s.at[idxb.at[st * 4 + j]],
                             ssems[st], add=True)

    def wait_gathers(st):
        for j in range(4):
            pltpu.make_async_copy(u2_hbm.at[idxa.at[j]],
                                  rows.at[st * 4 + j], gsems[st]).wait()

    def wait_scatters(st):
        for j in range(4):
            pltpu.make_async_copy(rows.at[st * 4 + j],
                                  acc_s.at[idxb.at[j]], ssems[st]).wait()

    nb2 = TCH // 8

    @pl.loop(0, nb2)
    def _(b):
        cb = start + b * 8
        pltpu.sync_copy(ts2d.at[pl.ds(cb, 8)], idxa)
        pltpu.sync_copy(td2d.at[pl.ds(cb, 8)], idxb)
        fire_gathers(0)
        fire_gathers(1)
        for st in range(2):
            wait_gathers(st)
            fire_scatters(st)
        for st in range(2):
            wait_scatters(st)

    plsc.subcore_barrier()
    pltpu.sync_copy(acc_s.at[pl.ds(s * SLC, SLC)],
                    g2_out.at[c, pl.ds(s * SLC, SLC)])


@functools.partial(
    pl.kernel,
    out_type=jax.ShapeDtypeStruct((2, ED_PAD, 64), jnp.float32),
    mesh=_mesh,
    compiler_params=_sc_params,
    scratch_types=[
        pltpu.VMEM((DCH_W, CH), jnp.int32),
        pltpu.VMEM((DCH_W, CH), jnp.int32),
        pltpu.VMEM((5, CH, 64), jnp.float32),
        pltpu.VMEM((5, CH, 64), jnp.float32),
        pltpu.SemaphoreType.DMA,
        pltpu.SemaphoreType.DMA,
        pltpu.SemaphoreType.DMA,
    ],
)
def _sc_stage3(e0_2d, e1_2d, z2_hbm, nr_out, idxa, idxb, rows0, rows1,
               g0sem, g1sem, stsem):
    c = lax.axis_index("c")
    s = lax.axis_index("s")
    wid = s * 2 + c
    start = wid * DCH_W
    nb = DCH_W // 5

    pltpu.sync_copy(e0_2d.at[pl.ds(start, DCH_W)], idxa)
    pltpu.sync_copy(e1_2d.at[pl.ds(start, DCH_W)], idxb)

    def fire_gathers(batch):
        for j in range(5):
            pltpu.async_copy(z2_hbm.at[idxa.at[batch * 5 + j]],
                             rows0.at[j], g0sem)
            pltpu.async_copy(z2_hbm.at[idxb.at[batch * 5 + j]],
                             rows1.at[j], g1sem)

    def wait_gathers():
        for j in range(5):
            pltpu.make_async_copy(z2_hbm.at[idxa.at[j]], rows0.at[j],
                                  g0sem).wait()
            pltpu.make_async_copy(z2_hbm.at[idxb.at[j]], rows1.at[j],
                                  g1sem).wait()

    fire_gathers(0)

    @pl.loop(0, nb)
    def _(b):
        wait_gathers()

        for j in range(5):
            sl = pl.ds((start + b * 5 + j) * CH, CH)
            pltpu.async_copy(rows0.at[j], nr_out.at[0, sl], stsem)
            pltpu.async_copy(rows1.at[j], nr_out.at[1, sl], stsem)

        for j in range(5):
            pltpu.make_async_copy(rows0.at[j], nr_out.at[0, pl.ds(0, CH)],
                                  stsem).wait()
            pltpu.make_async_copy(rows1.at[j], nr_out.at[1, pl.ds(0, CH)],
                                  stsem).wait()

        @pl.when(b < nb - 1)
        def _():
            fire_gathers(b + 1)


def _tc_prep_body(dega_ref, degb_ref, x_ref, dinv_ref, u_ref):
    deg = dega_ref[...] + degb_ref[...] + 1.0
    dinv = lax.rsqrt(jnp.maximum(deg, 1e-12))
    dinv_ref[...] = dinv
    u_ref[...] = x_ref[...] * dinv


def _tc_mid_body(dinv_ref, u_ref, g1a_ref, g1b_ref, W1_ref, b1_ref, W2_ref,
                 u2_ref):
    dinv = dinv_ref[...]
    u = u_ref[...]
    s1 = dinv * (g1a_ref[...] + g1b_ref[...] + u)
    z1 = jnp.maximum(s1 * W1_ref[...] + b1_ref[...], 0.0)
    h2 = jnp.dot(z1, W2_ref[...], preferred_element_type=jnp.float32)
    u2_ref[...] = h2 * dinv


def _tc_z2_body(dinv_ref, g2a_ref, g2b_ref, u2_ref, b2_ref, z2_ref):
    dinv = dinv_ref[...]
    agg = dinv * (g2a_ref[...] + g2b_ref[...] + u2_ref[...])
    z2_ref[...] = jnp.maximum(agg + b2_ref[...], 0.0)


def _tc_attr_body(ea_ref, L1a_ref, a1_ref):
    a1_ref[...] = jnp.dot(ea_ref[...], L1a_ref[...],
                          preferred_element_type=jnp.float32)


def _tc_dec_body(nr0_ref, nr1_ref, a1_ref, L1n_ref, L1b_ref, L2w_ref,
                 L2b_ref, out_ref):
    nr = nr0_ref[0] * nr1_ref[0]
    a = jnp.dot(nr, L1n_ref[...], preferred_element_type=jnp.float32)
    a += a1_ref[...]
    h = jnp.maximum(a + L1b_ref[...], 0.0)
    logits = jnp.dot(h, L2w_ref[...], preferred_element_type=jnp.float32)
    logits += L2b_ref[...]
    m = jnp.max(logits, axis=-1, keepdims=True)
    e = jnp.exp(logits - m)
    out_ref[...] = e / jnp.sum(e, axis=-1, keepdims=True)


_EB = 2048


def kernel(x, train_edge_index, edge_index, edge_attr,
           W1, b1, W2, b2, L1w, L1b, L2w, L2b):
    f32 = jnp.float32
    npad = E_PAD - E_TRAIN
    ts2d = jnp.concatenate(
        [train_edge_index[0],
         jnp.zeros((npad,), jnp.int32)]).reshape(NCHUNK, CH)
    td2d = jnp.concatenate(
        [train_edge_index[1],
         jnp.full((npad,), NP - 1, jnp.int32)]).reshape(NCHUNK, CH)
    e0_2d = jnp.pad(edge_index[0], (0, ED_PAD - E_DEC)).reshape(NDCH, CH)
    e1_2d = jnp.pad(edge_index[1], (0, ED_PAD - E_DEC)).reshape(NDCH, CH)
    xp = jnp.pad(x[:, 0], (0, NP - N))

    grid = (E_DEC + _EB - 1) // _EB
    a1 = pl.pallas_call(
        _tc_attr_body,
        grid=(grid,),
        in_specs=[
            pl.BlockSpec((_EB, HID), lambda i: (i, 0)),
            pl.BlockSpec((HID, 128), lambda i: (0, 0)),
        ],
        out_specs=pl.BlockSpec((_EB, 128), lambda i: (i, 0)),
        out_shape=jax.ShapeDtypeStruct((E_DEC, 128), f32),
    )(edge_attr, L1w[64:])

    deg = _sc_deg(td2d)

    dinv2, u = pl.pallas_call(
        _tc_prep_body,
        out_shape=(jax.ShapeDtypeStruct((NP, 1), f32),
                   jax.ShapeDtypeStruct((NP, 1), f32)),
    )(deg[0].reshape(NP, 1), deg[1].reshape(NP, 1), xp.reshape(NP, 1))

    g1 = _sc_g1(ts2d, td2d, u.reshape(NP))

    u2 = pl.pallas_call(
        _tc_mid_body,
        out_shape=jax.ShapeDtypeStruct((NP, 64), f32),
    )(dinv2, u, g1[0].reshape(NP, 1), g1[1].reshape(NP, 1),
      W1, b1.reshape(1, 128), W2)

    g2 = _sc_stage2(ts2d, td2d, u2)

    z2 = pl.pallas_call(
        _tc_z2_body,
        out_shape=jax.ShapeDtypeStruct((NP, 64), f32),
    )(dinv2, g2[0], g2[1], u2, b2.reshape(1, 64))

    nr = _sc_stage3(e0_2d, e1_2d, z2)

    out = pl.pallas_call(
        _tc_dec_body,
        grid=(grid,),
        in_specs=[
            pl.BlockSpec((1, _EB, 64), lambda i: (0, i, 0)),
            pl.BlockSpec((1, _EB, 64), lambda i: (1, i, 0)),
            pl.BlockSpec((_EB, 128), lambda i: (i, 0)),
            pl.BlockSpec((64, 128), lambda i: (0, 0)),
            pl.BlockSpec((1, 128), lambda i: (0, 0)),
            pl.BlockSpec((128, NC), lambda i: (0, 0)),
            pl.BlockSpec((1, NC), lambda i: (0, 0)),
        ],
        out_specs=pl.BlockSpec((_EB, NC), lambda i: (i, 0)),
        out_shape=jax.ShapeDtypeStruct((E_DEC, NC), f32),
    )(nr, nr, a1, L1w[:64], L1b.reshape(1, 128), L2w, L2b.reshape(1, NC))

    return out


# speedup vs baseline: 2.0880x; 1.0498x over previous
import functools

import jax
import jax.numpy as jnp
from jax import lax
from jax.experimental import pallas as pl
from jax.experimental.pallas import tpu as pltpu
from jax.experimental.pallas import tpu_sc as plsc

N = 10000
NP = 10240
E_TRAIN = 640000
E_PAD = 655360
E_DEC = 100000
ED_PAD = 102400
HID = 768
NC = 5
CH = 128
NCHUNK = E_PAD // CH
NCHUNK_HALF = NCHUNK // 2
TCH = NCHUNK_HALF // 16
NDCH = ED_PAD // CH
DCH_W = NDCH // 32
NSUB = 16
SLC = NP // NSUB

_mesh = plsc.VectorSubcoreMesh(core_axis_name="c", subcore_axis_name="s")
_sc_params = pltpu.CompilerParams(needs_layout_passes=False,
                                  use_tc_tiling_on_sc=False)


def _fill_const(ref, n16, value):
    @pl.loop(0, n16)
    def _(i):
        ref[pl.ds(i * 16, 16)] = jnp.full((16,), value, jnp.float32)


@functools.partial(
    pl.kernel,
    out_type=jax.ShapeDtypeStruct((2, NP), jnp.float32),
    mesh=_mesh,
    compiler_params=_sc_params,
    scratch_types=[
        pltpu.VMEM((8, CH), jnp.int32),
        pltpu.VMEM((CH,), jnp.float32),
        pltpu.VMEM((SLC,), jnp.float32),
        pltpu.VMEM_SHARED((NP,), jnp.float32),
        pltpu.SemaphoreType.DMA,
    ],
)
def _sc_deg(td2d, deg_out, idx2, ones_v, fill_v, deg_s, sem):
    c = lax.axis_index("c")
    s = lax.axis_index("s")
    base = s * SLC
    start = c * NCHUNK_HALF + s * TCH

    _fill_const(fill_v, SLC // 16, 0.0)
    pltpu.sync_copy(fill_v, deg_s.at[pl.ds(base, SLC)])
    _fill_const(ones_v, CH // 16, 1.0)
    plsc.subcore_barrier()

    @pl.loop(0, TCH // 8)
    def _(b):
        cb = start + b * 8
        pltpu.sync_copy(td2d.at[pl.ds(cb, 8)], idx2)
        descs = [pltpu.async_copy(ones_v, deg_s.at[idx2.at[j]], sem, add=True)
                 for j in range(8)]
        for d in descs:
            d.wait()

    plsc.subcore_barrier()
    pltpu.sync_copy(deg_s.at[pl.ds(base, SLC)], deg_out.at[c, pl.ds(base, SLC)])


@functools.partial(
    pl.kernel,
    out_type=jax.ShapeDtypeStruct((2, NP), jnp.float32),
    mesh=_mesh,
    compiler_params=_sc_params,
    scratch_types=[
        pltpu.VMEM((8, CH), jnp.int32),
        pltpu.VMEM((8, CH), jnp.int32),
        pltpu.VMEM((8, CH), jnp.float32),
        pltpu.VMEM((SLC,), jnp.float32),
        pltpu.VMEM((NP,), jnp.float32),
        pltpu.VMEM_SHARED((NP,), jnp.float32),
        pltpu.SemaphoreType.DMA,
    ],
)
def _sc_g1(ts2d, td2d, u_hbm, g1_out, idxa2, idxb2, valb, fill_v, tab, g1_s,
           sem):
    c = lax.axis_index("c")
    s = lax.axis_index("s")
    base = s * SLC
    start = c * NCHUNK_HALF + s * TCH

    _fill_const(fill_v, SLC // 16, 0.0)
    pltpu.sync_copy(fill_v, g1_s.at[pl.ds(base, SLC)])
    pltpu.sync_copy(u_hbm, tab)
    plsc.subcore_barrier()

    @pl.loop(0, TCH // 8)
    def _(b):
        cb = start + b * 8
        pltpu.sync_copy(ts2d.at[pl.ds(cb, 8)], idxa2)
        pltpu.sync_copy(td2d.at[pl.ds(cb, 8)], idxb2)

        @pl.loop(0, 8)
        def _(r):
            for k in range(CH // 16):
                sl = pl.ds(k * 16, 16)
                valb[r, sl] = plsc.load_gather(tab, [idxa2[r, sl]])

        descs = [pltpu.async_copy(valb.at[j], g1_s.at[idxb2.at[j]], sem,
                                  add=True)
                 for j in range(8)]
        for d in descs:
            d.wait()

    plsc.subcore_barrier()
    pltpu.sync_copy(g1_s.at[pl.ds(base, SLC)], g1_out.at[c, pl.ds(base, SLC)])


@functools.partial(
    pl.kernel,
    out_type=jax.ShapeDtypeStruct((2, NP, 64), jnp.float32),
    mesh=_mesh,
    compiler_params=_sc_params,
    scratch_types=[
        pltpu.VMEM((8, CH), jnp.int32),
        pltpu.VMEM((8, CH), jnp.int32),
        pltpu.VMEM((8, CH, 64), jnp.float32),
        pltpu.VMEM_SHARED((NP, 64), jnp.float32),
        pltpu.SemaphoreType.DMA,
        pltpu.SemaphoreType.DMA,
        pltpu.SemaphoreType.DMA,
        pltpu.SemaphoreType.DMA,
    ],
)
def _sc_stage2(ts2d, td2d, u2_hbm, g2_out, idxa, idxb, rows, acc_s,
               gsemA, gsemB, ssemA, ssemB):
    c = lax.axis_index("c")
    s = lax.axis_index("s")
    start = c * NCHUNK_HALF + s * TCH

    @pl.loop(0, CH)
    def _(r):
        for j in range(4):
            rows[0, r, pl.ds(j * 16, 16)] = jnp.zeros((16,), jnp.float32)

    for k in range(SLC // CH):
        pltpu.sync_copy(rows.at[0], acc_s.at[pl.ds(s * SLC + k * CH, CH)])
    plsc.subcore_barrier()

    gsems = (gsemA, gsemB)
    ssems = (ssemA, ssemB)

    def fire_gathers(st):
        for j in range(4):
            pltpu.async_copy(u2_hbm.at[idxa.at[st * 4 + j]],
                             rows.at[st * 4 + j], gsems[st])

    def fire_scatters(st):
        for j in range(4):
            pltpu.async_copy(rows.at[st * 4 + j],
                             acc_s.at[idxb.at[st * 4 + j]],
                             ssems[st], add=True)

    def wait_gathers(st):
        for j in range(4):
            pltpu.make_async_copy(u2_hbm.at[idxa.at[j]],
                                  rows.at[st * 4 + j], gsems[st]).wait()

    def wait_scatters(st):
        for j in range(4):
            pltpu.make_async_copy(rows.at[st * 4 + j],
                                  acc_s.at[idxb.at[j]], ssems[st]).wait()

    nb2 = TCH // 8

    @pl.loop(0, nb2)
    def _(b):
        cb = start + b * 8
        pltpu.sync_copy(ts2d.at[pl.ds(cb, 8)], idxa)
        pltpu.sync_copy(td2d.at[pl.ds(cb, 8)], idxb)
        fire_gathers(0)
        fire_gathers(1)
        for st in range(2):
            wait_gathers(st)
            fire_scatters(st)
        for st in range(2):
            wait_scatters(st)

    plsc.subcore_barrier()
    pltpu.sync_copy(acc_s.at[pl.ds(s * SLC, SLC)],
                    g2_out.at[c, pl.ds(s * SLC, SLC)])


@functools.partial(
    pl.kernel,
    out_type=jax.ShapeDtypeStruct((ED_PAD, 64), jnp.float32),
    mesh=_mesh,
    compiler_params=_sc_params,
    scratch_types=[
        pltpu.VMEM((DCH_W, CH), jnp.int32),
        pltpu.VMEM((DCH_W, CH), jnp.int32),
        pltpu.VMEM((5, CH, 64), jnp.float32),
        pltpu.VMEM((5, CH, 64), jnp.float32),
        pltpu.SemaphoreType.DMA,
        pltpu.SemaphoreType.DMA,
        pltpu.SemaphoreType.DMA,
    ],
)
def _sc_stage3(e0_2d, e1_2d, z2_hbm, nr_out, idxa, idxb, rows0, rows1,
               g0sem, g1sem, stsem):
    c = lax.axis_index("c")
    s = lax.axis_index("s")
    wid = s * 2 + c
    start = wid * DCH_W
    nb = DCH_W // 5

    pltpu.sync_copy(e0_2d.at[pl.ds(start, DCH_W)], idxa)
    pltpu.sync_copy(e1_2d.at[pl.ds(start, DCH_W)], idxb)

    def fire_gathers(batch):
        for j in range(5):
            pltpu.async_copy(z2_hbm.at[idxa.at[batch * 5 + j]],
                             rows0.at[j], g0sem)
            pltpu.async_copy(z2_hbm.at[idxb.at[batch * 5 + j]],
                             rows1.at[j], g1sem)

    def wait_gathers():
        for j in range(5):
            pltpu.make_async_copy(z2_hbm.at[idxa.at[j]], rows0.at[j],
                                  g0sem).wait()
            pltpu.make_async_copy(z2_hbm.at[idxb.at[j]], rows1.at[j],
                                  g1sem).wait()

    fire_gathers(0)

    @pl.loop(0, nb)
    def _(b):
        wait_gathers()

        @pl.loop(0, CH)
        def _(r):
            for j in range(5):
                for k in range(4):
                    sl = pl.ds(k * 16, 16)
                    rows0[j, r, sl] = rows0[j, r, sl] * rows1[j, r, sl]

        for j in range(5):
            pltpu.async_copy(
                rows0.at[j],
                nr_out.at[pl.ds((start + b * 5 + j) * CH, CH)], stsem)

        for j in range(5):
            pltpu.make_async_copy(rows0.at[j], nr_out.at[pl.ds(0, CH)],
                                  stsem).wait()

        @pl.when(b < nb - 1)
        def _():
            fire_gathers(b + 1)


def _tc_prep_body(dega_ref, degb_ref, x_ref, dinv_ref, u_ref):
    deg = dega_ref[...] + degb_ref[...] + 1.0
    dinv = lax.rsqrt(jnp.maximum(deg, 1e-12))
    dinv_ref[...] = dinv
    u_ref[...] = x_ref[...] * dinv


def _tc_mid_body(dinv_ref, u_ref, g1a_ref, g1b_ref, W1_ref, b1_ref, W2_ref,
                 u2_ref):
    dinv = dinv_ref[...]
    u = u_ref[...]
    s1 = dinv * (g1a_ref[...] + g1b_ref[...] + u)
    z1 = jnp.maximum(s1 * W1_ref[...] + b1_ref[...], 0.0)
    h2 = jnp.dot(z1, W2_ref[...], preferred_element_type=jnp.float32)
    u2_ref[...] = h2 * dinv


def _tc_z2_body(dinv_ref, g2a_ref, g2b_ref, u2_ref, b2_ref, z2_ref):
    dinv = dinv_ref[...]
    agg = dinv * (g2a_ref[...] + g2b_ref[...] + u2_ref[...])
    z2_ref[...] = jnp.maximum(agg + b2_ref[...], 0.0)


def _tc_attr_body(ea_ref, L1a_ref, a1_ref):
    a1_ref[...] = jnp.dot(ea_ref[...], L1a_ref[...],
                          preferred_element_type=jnp.float32)


def _tc_dec_body(nr_ref, a1_ref, L1n_ref, L1b_ref, L2w_ref,
                 L2b_ref, out_ref):
    a = jnp.dot(nr_ref[...], L1n_ref[...], preferred_element_type=jnp.float32)
    a += a1_ref[...]
    h = jnp.maximum(a + L1b_ref[...], 0.0)
    logits = jnp.dot(h, L2w_ref[...], preferred_element_type=jnp.float32)
    logits += L2b_ref[...]
    m = jnp.max(logits, axis=-1, keepdims=True)
    e = jnp.exp(logits - m)
    out_ref[...] = e / jnp.sum(e, axis=-1, keepdims=True)


_EB = 2048


def kernel(x, train_edge_index, edge_index, edge_attr,
           W1, b1, W2, b2, L1w, L1b, L2w, L2b):
    f32 = jnp.float32
    npad = E_PAD - E_TRAIN
    ts2d = jnp.concatenate(
        [train_edge_index[0],
         jnp.zeros((npad,), jnp.int32)]).reshape(NCHUNK, CH)
    td2d = jnp.concatenate(
        [train_edge_index[1],
         jnp.full((npad,), NP - 1, jnp.int32)]).reshape(NCHUNK, CH)
    e0_2d = jnp.pad(edge_index[0], (0, ED_PAD - E_DEC)).reshape(NDCH, CH)
    e1_2d = jnp.pad(edge_index[1], (0, ED_PAD - E_DEC)).reshape(NDCH, CH)
    xp = jnp.pad(x[:, 0], (0, NP - N))

    grid = (E_DEC + _EB - 1) // _EB
    a1 = pl.pallas_call(
        _tc_attr_body,
        grid=(grid,),
        in_specs=[
            pl.BlockSpec((_EB, HID), lambda i: (i, 0)),
            pl.BlockSpec((HID, 128), lambda i: (0, 0)),
        ],
        out_specs=pl.BlockSpec((_EB, 128), lambda i: (i, 0)),
        out_shape=jax.ShapeDtypeStruct((E_DEC, 128), f32),
    )(edge_attr, L1w[64:])

    deg = _sc_deg(td2d)

    dinv2, u = pl.pallas_call(
        _tc_prep_body,
        out_shape=(jax.ShapeDtypeStruct((NP, 1), f32),
                   jax.ShapeDtypeStruct((NP, 1), f32)),
    )(deg[0].reshape(NP, 1), deg[1].reshape(NP, 1), xp.reshape(NP, 1))

    g1 = _sc_g1(ts2d, td2d, u.reshape(NP))

    u2 = pl.pallas_call(
        _tc_mid_body,
        out_shape=jax.ShapeDtypeStruct((NP, 64), f32),
    )(dinv2, u, g1[0].reshape(NP, 1), g1[1].reshape(NP, 1),
      W1, b1.reshape(1, 128), W2)

    g2 = _sc_stage2(ts2d, td2d, u2)

    z2 = pl.pallas_call(
        _tc_z2_body,
        out_shape=jax.ShapeDtypeStruct((NP, 64), f32),
    )(dinv2, g2[0], g2[1], u2, b2.reshape(1, 64))

    nr = _sc_stage3(e0_2d, e1_2d, z2)

    out = pl.pallas_call(
        _tc_dec_body,
        grid=(grid,),
        in_specs=[
            pl.BlockSpec((_EB, 64), lambda i: (i, 0)),
            pl.BlockSpec((_EB, 128), lambda i: (i, 0)),
            pl.BlockSpec((64, 128), lambda i: (0, 0)),
            pl.BlockSpec((1, 128), lambda i: (0, 0)),
            pl.BlockSpec((128, NC), lambda i: (0, 0)),
            pl.BlockSpec((1, NC), lambda i: (0, 0)),
        ],
        out_specs=pl.BlockSpec((_EB, NC), lambda i: (i, 0)),
        out_shape=jax.ShapeDtypeStruct((E_DEC, NC), f32),
    )(nr, a1, L1w[:64], L1b.reshape(1, 128), L2w, L2b.reshape(1, NC))

    return out
